# Initial kernel scaffold; baseline (speedup 1.0000x reference)
import jax
import jax.numpy as jnp
from jax.experimental import pallas as pl


def kernel(x, enc_w, enc_b, dec_w, dec_b):
    raise NotImplementedError("write your pallas kernel here")


# speedup vs baseline: 7.7062x; 7.7062x over previous
import functools

import jax
import jax.numpy as jnp
from jax.experimental import pallas as pl
from jax.experimental.pallas import tpu as pltpu

TOPK = 64


def _enc_body(x_ref, w_ref, b_ref, db_ref, out_ref):
    xb = (x_ref[...] - db_ref[...]).astype(jnp.bfloat16)
    aw = jax.lax.dot_general(
        xb, w_ref[...], (((1,), (1,)), ((), ())),
        preferred_element_type=jnp.float32,
    )
    out_ref[...] = aw + b_ref[...]


def _key_of(a):
    u = jax.lax.bitcast_convert_type(a, jnp.int32)
    return jnp.where(u >= 0, u, u ^ jnp.int32(0x7FFFFFFF))


def _thresh_body(a_ref, t_ref, *, k):
    a = a_ref[...]
    ks = _key_of(a)
    kk = ks ^ jnp.int32(-2147483648)
    bm = a.shape[0]
    prefix0 = jnp.zeros((bm, 1), jnp.int32)
    krem0 = jnp.full((bm, 1), k, jnp.int32)

    def body(i, carry):
        prefix, krem = carry
        b = 31 - i
        elem_hi = jax.lax.shift_right_logical(kk, b)
        cand = jax.lax.shift_right_logical(prefix, b) | jnp.int32(1)
        m = elem_hi == cand
        cnt = jnp.sum(m.astype(jnp.int32), axis=1, keepdims=True)
        take = cnt >= krem
        bit = jax.lax.shift_left(jnp.int32(1), b)
        prefix = jnp.where(take, prefix | bit, prefix)
        krem = jnp.where(take, krem, krem - cnt)
        return prefix, krem

    prefix, _ = jax.lax.fori_loop(0, 32, body, (prefix0, krem0))
    t_ref[...] = prefix ^ jnp.int32(-2147483648)


def _dec_body(a_ref, t_ref, w_ref, db_ref, f_ref, xhat_ref, acc_ref, *, nk):
    j = pl.program_id(1)
    a = a_ref[...]
    ks = _key_of(a)
    m = ks >= t_ref[...]
    f = jnp.maximum(jnp.where(m, a, 0.0), 0.0)
    f_ref[...] = f
    partial = jax.lax.dot_general(
        f.astype(jnp.bfloat16), w_ref[...], (((1,), (1,)), ((), ())),
        preferred_element_type=jnp.float32,
    )

    @pl.when(j == 0)
    def _():
        acc_ref[...] = partial

    @pl.when(j > 0)
    def _():
        acc_ref[...] += partial

    @pl.when(j == nk - 1)
    def _():
        xhat_ref[...] = acc_ref[...] + db_ref[...]


def kernel(x, enc_w, enc_b, dec_w, dec_b):
    ntok, vec = x.shape
    lat = enc_w.shape[0]

    bm = 256 if ntok % 256 == 0 else ntok
    bn = 2048 if lat % 2048 == 0 else lat
    n_lat = lat // bn

    enc_b2 = enc_b.reshape(1, lat)
    dec_b2 = dec_b.reshape(1, vec)
    enc_wh = enc_w.astype(jnp.bfloat16)
    dec_wh = dec_w.astype(jnp.bfloat16)

    a = pl.pallas_call(
        _enc_body,
        grid=(ntok // bm, n_lat),
        in_specs=[
            pl.BlockSpec((bm, vec), lambda i, j: (i, 0)),
            pl.BlockSpec((bn, vec), lambda i, j: (j, 0)),
            pl.BlockSpec((1, bn), lambda i, j: (0, j)),
            pl.BlockSpec((1, vec), lambda i, j: (0, 0)),
        ],
        out_specs=pl.BlockSpec((bm, bn), lambda i, j: (i, j)),
        out_shape=jax.ShapeDtypeStruct((ntok, lat), jnp.float32),
    )(x, enc_wh, enc_b2, dec_b2)

    bmt = 256 if ntok % 256 == 0 else ntok
    tkeys = pl.pallas_call(
        functools.partial(_thresh_body, k=TOPK),
        grid=(ntok // bmt,),
        in_specs=[pl.BlockSpec((bmt, lat), lambda i: (i, 0))],
        out_specs=pl.BlockSpec((bmt, 1), lambda i: (i, 0)),
        out_shape=jax.ShapeDtypeStruct((ntok, 1), jnp.int32),
    )(a)

    bk = 2048 if lat % 2048 == 0 else lat
    nk = lat // bk
    f, xhat = pl.pallas_call(
        functools.partial(_dec_body, nk=nk),
        grid=(ntok // bm, nk),
        in_specs=[
            pl.BlockSpec((bm, bk), lambda i, j: (i, j)),
            pl.BlockSpec((bm, 1), lambda i, j: (i, 0)),
            pl.BlockSpec((vec, bk), lambda i, j: (0, j)),
            pl.BlockSpec((1, vec), lambda i, j: (0, 0)),
        ],
        out_specs=[
            pl.BlockSpec((bm, bk), lambda i, j: (i, j)),
            pl.BlockSpec((bm, vec), lambda i, j: (i, 0)),
        ],
        out_shape=[
            jax.ShapeDtypeStruct((ntok, lat), jnp.float32),
            jax.ShapeDtypeStruct((ntok, vec), jnp.float32),
        ],
        scratch_shapes=[pltpu.VMEM((bm, vec), jnp.float32)],
    )(a, tkeys, dec_wh, dec_b2)

    return (f, xhat)
